# baseline (speedup 1.0000x reference)
import functools

import jax
import jax.numpy as jnp
from jax import lax
from jax.experimental import pallas as pl
from jax.experimental.pallas import tpu as pltpu
from jax.experimental.pallas import tpu_sc as plsc

_N = 4194304
_LAMB = 0.5
_MARGIN = 1.0

_NC = 2
_NS = 16
_NW = _NC * _NS
_L = 16
_PER_W = _N // _NW
_CHUNK = 16384
_STEPS = _PER_W // _CHUNK
_VSTEPS = _CHUNK // _L


def _tec_body(t_hbm, s_hbm, d_hbm, out_hbm, t_v, s_v, d_v, acc_v):
    wid = lax.axis_index("s") * _NC + lax.axis_index("c")
    base = wid * _PER_W

    def chunk_body(c, acc):
        off = base + c * _CHUNK
        pltpu.sync_copy(t_hbm.at[pl.ds(off, _CHUNK)], t_v)
        pltpu.sync_copy(s_hbm.at[pl.ds(off, _CHUNK)], s_v)
        pltpu.sync_copy(d_hbm.at[pl.ds(off, _CHUNK)], d_v)

        def vec_body(i, a):
            sl = pl.ds(i * _L, _L)
            t = t_v[sl]
            s = s_v[sl]
            d = d_v[sl]
            diff = s - d
            nc = jnp.maximum(0.0, _LAMB * (diff - _MARGIN))
            r = jnp.where(t <= 0.5, nc, diff)
            return a + r * r

        return lax.fori_loop(0, _VSTEPS, vec_body, acc)

    acc = lax.fori_loop(0, _STEPS, chunk_body, jnp.zeros((_L,), jnp.float32))
    acc_v[...] = acc
    pltpu.sync_copy(acc_v, out_hbm.at[pl.ds(wid * _L, _L)])


@functools.partial(
    pl.kernel,
    out_type=jax.ShapeDtypeStruct((_NW * _L,), jnp.float32),
    mesh=plsc.VectorSubcoreMesh(core_axis_name="c", subcore_axis_name="s"),
    scratch_types=[
        pltpu.VMEM((_CHUNK,), jnp.float32),
        pltpu.VMEM((_CHUNK,), jnp.float32),
        pltpu.VMEM((_CHUNK,), jnp.float32),
        pltpu.VMEM((_L,), jnp.float32),
    ],
)
def _sc_partials(t_hbm, s_hbm, d_hbm, out_hbm, t_v, s_v, d_v, acc_v):
    _tec_body(t_hbm, s_hbm, d_hbm, out_hbm, t_v, s_v, d_v, acc_v)


def kernel(inputs, targets, scan_t, diag_t):
    del inputs
    partials = _sc_partials(targets, scan_t, diag_t)
    return jnp.sum(partials) / _N


# speedup vs baseline: 1.9028x; 1.9028x over previous
import functools

import jax
import jax.numpy as jnp
from jax import lax
from jax.experimental import pallas as pl
from jax.experimental.pallas import tpu as pltpu
from jax.experimental.pallas import tpu_sc as plsc

_N = 4194304
_LAMB = 0.5
_MARGIN = 1.0

_NC = 2
_NS = 16
_NW = _NC * _NS
_L = 16
_PER_W = _N // _NW
_CHUNK = 16384
_STEPS = _PER_W // _CHUNK
_UNROLL = 8
_ACCS = 4
_VSTEPS = _CHUNK // (_L * _UNROLL)


def _tec_body(t_hbm, s_hbm, d_hbm, out_hbm, bufs, sems, acc_v):
    wid = lax.axis_index("s") * _NC + lax.axis_index("c")
    base = wid * _PER_W

    def issue(c, b):
        off = base + c * _CHUNK
        return [
            pltpu.async_copy(t_hbm.at[pl.ds(off, _CHUNK)], bufs[b][0], sems[b][0]),
            pltpu.async_copy(s_hbm.at[pl.ds(off, _CHUNK)], bufs[b][1], sems[b][1]),
            pltpu.async_copy(d_hbm.at[pl.ds(off, _CHUNK)], bufs[b][2], sems[b][2]),
        ]

    def compute(t_v, s_v, d_v, accs):
        def vec_body(i, a):
            a = list(a)
            for u in range(_UNROLL):
                sl = pl.ds((i * _UNROLL + u) * _L, _L)
                t = t_v[sl]
                s = s_v[sl]
                d = d_v[sl]
                diff = s - d
                nc = jnp.maximum(0.0, diff * _LAMB - (_LAMB * _MARGIN))
                r = jnp.where(t <= 0.5, nc, diff)
                a[u % _ACCS] = a[u % _ACCS] + r * r
            return tuple(a)

        return lax.fori_loop(0, _VSTEPS, vec_body, accs)

    zero = jnp.zeros((_L,), jnp.float32)
    accs = (zero,) * _ACCS
    pend = issue(0, 0)
    for c in range(_STEPS):
        if c + 1 < _STEPS:
            nxt = issue(c + 1, (c + 1) % 2)
        else:
            nxt = []
        for h in pend:
            h.wait()
        b = c % 2
        accs = compute(bufs[b][0], bufs[b][1], bufs[b][2], accs)
        pend = nxt

    acc = accs[0]
    for a in accs[1:]:
        acc = acc + a
    acc_v[...] = acc
    pltpu.sync_copy(acc_v, out_hbm.at[pl.ds(wid * _L, _L)])


@functools.partial(
    pl.kernel,
    out_type=jax.ShapeDtypeStruct((_NW * _L,), jnp.float32),
    mesh=plsc.VectorSubcoreMesh(core_axis_name="c", subcore_axis_name="s"),
    scratch_types=[
        pltpu.VMEM((_CHUNK,), jnp.float32),
        pltpu.VMEM((_CHUNK,), jnp.float32),
        pltpu.VMEM((_CHUNK,), jnp.float32),
        pltpu.VMEM((_CHUNK,), jnp.float32),
        pltpu.VMEM((_CHUNK,), jnp.float32),
        pltpu.VMEM((_CHUNK,), jnp.float32),
        pltpu.VMEM((_L,), jnp.float32),
        pltpu.SemaphoreType.DMA,
        pltpu.SemaphoreType.DMA,
        pltpu.SemaphoreType.DMA,
        pltpu.SemaphoreType.DMA,
        pltpu.SemaphoreType.DMA,
        pltpu.SemaphoreType.DMA,
    ],
)
def _sc_partials(t_hbm, s_hbm, d_hbm, out_hbm,
                 t0, s0, d0, t1, s1, d1, acc_v,
                 st0, ss0, sd0, st1, ss1, sd1):
    bufs = [(t0, s0, d0), (t1, s1, d1)]
    sems = [(st0, ss0, sd0), (st1, ss1, sd1)]
    _tec_body(t_hbm, s_hbm, d_hbm, out_hbm, bufs, sems, acc_v)


def kernel(inputs, targets, scan_t, diag_t):
    del inputs
    partials = _sc_partials(targets, scan_t, diag_t)
    return jnp.sum(partials) / _N
